# Initial kernel scaffold; baseline (speedup 1.0000x reference)
import jax
import jax.numpy as jnp
from jax.experimental import pallas as pl


def kernel(x, edge_index, batch, W_in, b_in, Wc0, bc0, Wc1, bc1, Wc2, bc2, Wh1, bh1, Wh2, bh2):
    raise NotImplementedError("write your pallas kernel here")


# speedup vs baseline: 13.2607x; 13.2607x over previous
import functools

import jax
import jax.numpy as jnp
from jax import lax
from jax.experimental import pallas as pl
from jax.experimental.pallas import tpu as pltpu
from jax.experimental.pallas import tpu_sc as plsc

N = 10000
D = 128
G = 64
OUT = 10
E = 320000

NC = 2
NS = 16
NW = NC * NS
CL = 128
CH = 79
EPAD = NW * CH * CL
NP = 10240
RPW = NP // NS

_R = 2048
_GRID = NP // _R


def _sc_agg_body(width, u_hbm, src_hbm, dst_hbm, p_hbm,
                 srcv, dstv, rows, stage, acc, sem):
    c = lax.axis_index("c")
    s = lax.axis_index("s")
    w = c * NS + s

    def _seed(part, _):
        base = s * RPW + part * CL
        pltpu.sync_copy(u_hbm.at[pl.ds(base, CL)], stage)
        pltpu.sync_copy(stage, acc.at[pl.ds(base, CL)])
        return ()
    lax.fori_loop(0, RPW // CL, _seed, ())
    plsc.subcore_barrier()

    def _chunk(j, _):
        pltpu.sync_copy(src_hbm.at[w, j], srcv)
        pltpu.sync_copy(dst_hbm.at[w, j], dstv)
        pltpu.async_copy(u_hbm.at[srcv], rows, sem).wait()
        pltpu.sync_copy(rows, acc.at[dstv], add=True)
        return ()
    lax.fori_loop(0, CH, _chunk, ())

    plsc.subcore_barrier()

    def _drain(part, _):
        base = s * RPW + part * CL
        pltpu.sync_copy(acc.at[pl.ds(base, CL)], stage)
        pltpu.sync_copy(stage, p_hbm.at[c, pl.ds(base, CL)])
        return ()
    lax.fori_loop(0, RPW // CL, _drain, ())


def _sc_deg_body(ones_hbm, dst_hbm, p_hbm, dstv, ones_t, stage, acc):
    c = lax.axis_index("c")
    s = lax.axis_index("s")
    w = c * NS + s

    def _seed(part, _):
        base = s * RPW + part * CL
        pltpu.sync_copy(ones_hbm.at[pl.ds(base, CL)], stage)
        pltpu.sync_copy(stage, acc.at[pl.ds(base, CL)])
        return ()
    lax.fori_loop(0, RPW // CL, _seed, ())
    pltpu.sync_copy(ones_hbm.at[pl.ds(0, CL)], ones_t)
    plsc.subcore_barrier()

    def _chunk(j, _):
        pltpu.sync_copy(dst_hbm.at[w, j], dstv)
        pltpu.sync_copy(ones_t, acc.at[dstv], add=True)
        return ()
    lax.fori_loop(0, CH, _chunk, ())

    plsc.subcore_barrier()

    def _drain(part, _):
        base = s * RPW + part * CL
        pltpu.sync_copy(acc.at[pl.ds(base, CL)], stage)
        pltpu.sync_copy(stage, p_hbm.at[c, pl.ds(base, CL)])
        return ()
    lax.fori_loop(0, RPW // CL, _drain, ())


@functools.cache
def _sc_mesh():
    return plsc.VectorSubcoreMesh(core_axis_name="c", subcore_axis_name="s",
                                  num_cores=NC, num_subcores=NS)


@functools.cache
def _sc_deg():
    return pl.kernel(
        _sc_deg_body,
        out_type=jax.ShapeDtypeStruct((NC, NP, 16), jnp.float32),
        mesh=_sc_mesh(),
        scratch_types=[
            pltpu.VMEM((CL,), jnp.int32),
            pltpu.VMEM((CL, 16), jnp.float32),
            pltpu.VMEM((CL, 16), jnp.float32),
            pltpu.VMEM_SHARED((NP, 16), jnp.float32),
        ],
    )


@functools.cache
def _sc_agg(width):
    return pl.kernel(
        functools.partial(_sc_agg_body, width),
        out_type=jax.ShapeDtypeStruct((NC, NP, width), jnp.float32),
        mesh=_sc_mesh(),
        scratch_types=[
            pltpu.VMEM((CL,), jnp.int32),
            pltpu.VMEM((CL,), jnp.int32),
            pltpu.VMEM((CL, width), jnp.float32),
            pltpu.VMEM((CL, width), jnp.float32),
            pltpu.VMEM_SHARED((NP, width), jnp.float32),
            pltpu.SemaphoreType.DMA,
        ],
    )


def _tc0_body(x_ref, d0_ref, d1_ref, win_ref, bin_ref, wc0_ref, u0_ref, dis_ref):
    deg = d0_ref[:, :1] + d1_ref[:, :1] - 1.0
    dis = lax.rsqrt(deg)
    dis_ref[...] = dis
    h0 = jnp.dot(x_ref[...], win_ref[...],
                 preferred_element_type=jnp.float32) + bin_ref[...]
    u0_ref[...] = dis * jnp.dot(h0, wc0_ref[...],
                                preferred_element_type=jnp.float32)


def _tcmid_body(p0_ref, p1_ref, up_ref, dis_ref, b_ref, w_ref, u_ref):
    dis = dis_ref[...]
    h = jnp.maximum(
        dis * (p0_ref[...] + p1_ref[...] - up_ref[...]) + b_ref[...], 0.0)
    u_ref[...] = dis * jnp.dot(h, w_ref[...],
                               preferred_element_type=jnp.float32)


def _tcfinal_body(p0_ref, p1_ref, up_ref, dis_ref, b_ref, bat_ref,
                  wh1_ref, bh1_ref, wh2_ref, bh2_ref, out_ref, sums, cnts):
    i = pl.program_id(0)

    @pl.when(i == 0)
    def _():
        sums[...] = jnp.zeros_like(sums)
        cnts[...] = jnp.zeros_like(cnts)

    dis = dis_ref[...]
    h = jnp.maximum(
        dis * (p0_ref[...] + p1_ref[...] - up_ref[...]) + b_ref[...], 0.0)
    gids = lax.broadcasted_iota(jnp.int32, (G, 1), 0)
    mask = (bat_ref[...] == gids).astype(jnp.float32)
    sums[...] += jnp.dot(mask, h, preferred_element_type=jnp.float32)
    cnts[...] += jnp.sum(mask, axis=1, keepdims=True)

    @pl.when(i == _GRID - 1)
    def _():
        g = sums[...] / jnp.maximum(cnts[...], 1.0)
        z = jnp.maximum(jnp.dot(g, wh1_ref[...],
                                preferred_element_type=jnp.float32)
                        + bh1_ref[...], 0.0)
        out_ref[...] = jnp.dot(z, wh2_ref[...],
                               preferred_element_type=jnp.float32) + bh2_ref[...]


def _row_spec(cols):
    return pl.BlockSpec((_R, cols), lambda i: (i, 0))


def _const_spec(shape):
    return pl.BlockSpec(shape, lambda i: tuple(0 for _ in shape))


_tc0 = pl.pallas_call(
    _tc0_body,
    grid=(_GRID,),
    in_specs=[
        _row_spec(D), _row_spec(16), _row_spec(16),
        _const_spec((D, D)), _const_spec((1, D)), _const_spec((D, D)),
    ],
    out_specs=[_row_spec(D), _row_spec(1)],
    out_shape=[
        jax.ShapeDtypeStruct((NP, D), jnp.float32),
        jax.ShapeDtypeStruct((NP, 1), jnp.float32),
    ],
)

_tcmid = pl.pallas_call(
    _tcmid_body,
    grid=(_GRID,),
    in_specs=[
        _row_spec(D), _row_spec(D), _row_spec(D), _row_spec(1),
        _const_spec((1, D)), _const_spec((D, D)),
    ],
    out_specs=_row_spec(D),
    out_shape=jax.ShapeDtypeStruct((NP, D), jnp.float32),
)

_tcfinal = pl.pallas_call(
    _tcfinal_body,
    grid=(_GRID,),
    in_specs=[
        _row_spec(D), _row_spec(D), _row_spec(D), _row_spec(1),
        _const_spec((1, D)), pl.BlockSpec((1, _R), lambda i: (0, i)),
        _const_spec((D, D // 2)), _const_spec((1, D // 2)),
        _const_spec((D // 2, OUT)), _const_spec((1, OUT)),
    ],
    out_specs=_const_spec((G, OUT)),
    out_shape=jax.ShapeDtypeStruct((G, OUT), jnp.float32),
    scratch_shapes=[
        pltpu.VMEM((G, D), jnp.float32),
        pltpu.VMEM((G, 1), jnp.float32),
    ],
)


@jax.jit
def kernel(x, edge_index, batch, W_in, b_in, Wc0, bc0, Wc1, bc1, Wc2, bc2,
           Wh1, bh1, Wh2, bh2):
    pad_idx = (N + (jnp.arange(EPAD - E, dtype=jnp.int32) % (NP - N)))
    src = jnp.concatenate([edge_index[0], pad_idx]).reshape(NW, CH, CL)
    dst = jnp.concatenate([edge_index[1], pad_idx]).reshape(NW, CH, CL)
    x_pad = jnp.pad(x, ((0, NP - N), (0, 0)))
    bat = jnp.pad(batch, (0, NP - N), constant_values=G).reshape(1, NP)
    ones16 = jnp.ones((NP, 16), jnp.float32)

    d = _sc_deg()(ones16, dst)
    u0, dis = _tc0(x_pad, d[0], d[1], W_in, b_in.reshape(1, D), Wc0)

    p = _sc_agg(D)(u0, src, dst)
    u1 = _tcmid(p[0], p[1], u0, dis, bc0.reshape(1, D), Wc1)
    p = _sc_agg(D)(u1, src, dst)
    u2 = _tcmid(p[0], p[1], u1, dis, bc1.reshape(1, D), Wc2)
    p = _sc_agg(D)(u2, src, dst)
    logits = _tcfinal(p[0], p[1], u2, dis, bc2.reshape(1, D), bat,
                      Wh1, bh1.reshape(1, D // 2), Wh2, bh2.reshape(1, OUT))
    return logits


# speedup vs baseline: 18.2449x; 1.3759x over previous
import functools

import jax
import jax.numpy as jnp
from jax import lax
from jax.experimental import pallas as pl
from jax.experimental.pallas import tpu as pltpu
from jax.experimental.pallas import tpu_sc as plsc

N = 10000
D = 128
G = 64
OUT = 10
E = 320000

NC = 2
NS = 16
NW = NC * NS
CL = 128
CH = 80
EPAD = NW * CH * CL
NP = 10240
RPW = NP // NS

_R = 2048
_GRID = NP // _R


def _sc_agg_body(width, u_hbm, src_hbm, dst_hbm, p_hbm,
                 s0, d0, s1, d1, r0, r1, acc, gs0, gs1):
    c = lax.axis_index("c")
    s = lax.axis_index("s")
    w = c * NS + s

    def _seed(part, _):
        base = s * RPW + part * CL
        pltpu.sync_copy(u_hbm.at[pl.ds(base, CL)], r0)
        pltpu.sync_copy(r0, acc.at[pl.ds(base, CL)])
        return ()
    lax.fori_loop(0, RPW // CL, _seed, ())
    plsc.subcore_barrier()

    pltpu.sync_copy(src_hbm.at[w, 0], s0)
    pltpu.sync_copy(dst_hbm.at[w, 0], d0)

    def _pair(k, _):
        j = 2 * k
        c0 = pltpu.async_copy(u_hbm.at[s0], r0, gs0)
        pltpu.sync_copy(src_hbm.at[w, j + 1], s1)
        pltpu.sync_copy(dst_hbm.at[w, j + 1], d1)
        c1 = pltpu.async_copy(u_hbm.at[s1], r1, gs1)
        c0.wait()
        pltpu.sync_copy(r0, acc.at[d0], add=True)
        jn = jnp.minimum(j + 2, CH - 1)
        pltpu.sync_copy(src_hbm.at[w, jn], s0)
        pltpu.sync_copy(dst_hbm.at[w, jn], d0)
        c1.wait()
        pltpu.sync_copy(r1, acc.at[d1], add=True)
        return ()
    lax.fori_loop(0, CH // 2, _pair, ())

    plsc.subcore_barrier()

    def _drain(part, _):
        base = s * RPW + part * CL
        pltpu.sync_copy(acc.at[pl.ds(base, CL)], r0)
        pltpu.sync_copy(r0, p_hbm.at[c, pl.ds(base, CL)])
        return ()
    lax.fori_loop(0, RPW // CL, _drain, ())


def _sc_deg_body(ones_hbm, dst_hbm, p_hbm, dstv, ones_t, stage, acc):
    c = lax.axis_index("c")
    s = lax.axis_index("s")
    w = c * NS + s

    def _seed(part, _):
        base = s * RPW + part * CL
        pltpu.sync_copy(ones_hbm.at[pl.ds(base, CL)], stage)
        pltpu.sync_copy(stage, acc.at[pl.ds(base, CL)])
        return ()
    lax.fori_loop(0, RPW // CL, _seed, ())
    pltpu.sync_copy(ones_hbm.at[pl.ds(0, CL)], ones_t)
    plsc.subcore_barrier()

    def _chunk(j, _):
        pltpu.sync_copy(dst_hbm.at[w, j], dstv)
        pltpu.sync_copy(ones_t, acc.at[dstv], add=True)
        return ()
    lax.fori_loop(0, CH, _chunk, ())

    plsc.subcore_barrier()

    def _drain(part, _):
        base = s * RPW + part * CL
        pltpu.sync_copy(acc.at[pl.ds(base, CL)], stage)
        pltpu.sync_copy(stage, p_hbm.at[c, pl.ds(base, CL)])
        return ()
    lax.fori_loop(0, RPW // CL, _drain, ())


@functools.cache
def _sc_mesh():
    return plsc.VectorSubcoreMesh(core_axis_name="c", subcore_axis_name="s",
                                  num_cores=NC, num_subcores=NS)


@functools.cache
def _sc_deg():
    return pl.kernel(
        _sc_deg_body,
        out_type=jax.ShapeDtypeStruct((NC, NP, 16), jnp.float32),
        mesh=_sc_mesh(),
        scratch_types=[
            pltpu.VMEM((CL,), jnp.int32),
            pltpu.VMEM((CL, 16), jnp.float32),
            pltpu.VMEM((CL, 16), jnp.float32),
            pltpu.VMEM_SHARED((NP, 16), jnp.float32),
        ],
    )


@functools.cache
def _sc_agg(width):
    return pl.kernel(
        functools.partial(_sc_agg_body, width),
        out_type=jax.ShapeDtypeStruct((NC, NP, width), jnp.float32),
        mesh=_sc_mesh(),
        scratch_types=[
            pltpu.VMEM((CL,), jnp.int32),
            pltpu.VMEM((CL,), jnp.int32),
            pltpu.VMEM((CL,), jnp.int32),
            pltpu.VMEM((CL,), jnp.int32),
            pltpu.VMEM((CL, width), jnp.float32),
            pltpu.VMEM((CL, width), jnp.float32),
            pltpu.VMEM_SHARED((NP, width), jnp.float32),
            pltpu.SemaphoreType.DMA,
            pltpu.SemaphoreType.DMA,
        ],
    )


def _tc0_body(x_ref, d0_ref, d1_ref, win_ref, bin_ref, wc0_ref, u0_ref, dis_ref):
    deg = d0_ref[:, :1] + d1_ref[:, :1] - 1.0
    dis = lax.rsqrt(deg)
    dis_ref[...] = dis
    h0 = jnp.dot(x_ref[...], win_ref[...],
                 preferred_element_type=jnp.float32) + bin_ref[...]
    u0_ref[...] = dis * jnp.dot(h0, wc0_ref[...],
                                preferred_element_type=jnp.float32)


def _tcmid_body(p0_ref, p1_ref, up_ref, dis_ref, b_ref, w_ref, u_ref):
    dis = dis_ref[...]
    h = jnp.maximum(
        dis * (p0_ref[...] + p1_ref[...] - up_ref[...]) + b_ref[...], 0.0)
    u_ref[...] = dis * jnp.dot(h, w_ref[...],
                               preferred_element_type=jnp.float32)


def _tcfinal_body(p0_ref, p1_ref, up_ref, dis_ref, b_ref, bat_ref,
                  wh1_ref, bh1_ref, wh2_ref, bh2_ref, out_ref, sums, cnts):
    i = pl.program_id(0)

    @pl.when(i == 0)
    def _():
        sums[...] = jnp.zeros_like(sums)
        cnts[...] = jnp.zeros_like(cnts)

    dis = dis_ref[...]
    h = jnp.maximum(
        dis * (p0_ref[...] + p1_ref[...] - up_ref[...]) + b_ref[...], 0.0)
    gids = lax.broadcasted_iota(jnp.int32, (G, 1), 0)
    mask = (bat_ref[...] == gids).astype(jnp.float32)
    sums[...] += jnp.dot(mask, h, preferred_element_type=jnp.float32)
    cnts[...] += jnp.sum(mask, axis=1, keepdims=True)

    @pl.when(i == _GRID - 1)
    def _():
        g = sums[...] / jnp.maximum(cnts[...], 1.0)
        z = jnp.maximum(jnp.dot(g, wh1_ref[...],
                                preferred_element_type=jnp.float32)
                        + bh1_ref[...], 0.0)
        out_ref[...] = jnp.dot(z, wh2_ref[...],
                               preferred_element_type=jnp.float32) + bh2_ref[...]


def _row_spec(cols):
    return pl.BlockSpec((_R, cols), lambda i: (i, 0))


def _const_spec(shape):
    return pl.BlockSpec(shape, lambda i: tuple(0 for _ in shape))


_tc0 = pl.pallas_call(
    _tc0_body,
    grid=(_GRID,),
    in_specs=[
        _row_spec(D), _row_spec(16), _row_spec(16),
        _const_spec((D, D)), _const_spec((1, D)), _const_spec((D, D)),
    ],
    out_specs=[_row_spec(D), _row_spec(1)],
    out_shape=[
        jax.ShapeDtypeStruct((NP, D), jnp.float32),
        jax.ShapeDtypeStruct((NP, 1), jnp.float32),
    ],
)

_tcmid = pl.pallas_call(
    _tcmid_body,
    grid=(_GRID,),
    in_specs=[
        _row_spec(D), _row_spec(D), _row_spec(D), _row_spec(1),
        _const_spec((1, D)), _const_spec((D, D)),
    ],
    out_specs=_row_spec(D),
    out_shape=jax.ShapeDtypeStruct((NP, D), jnp.float32),
)

_tcfinal = pl.pallas_call(
    _tcfinal_body,
    grid=(_GRID,),
    in_specs=[
        _row_spec(D), _row_spec(D), _row_spec(D), _row_spec(1),
        _const_spec((1, D)), pl.BlockSpec((1, _R), lambda i: (0, i)),
        _const_spec((D, D // 2)), _const_spec((1, D // 2)),
        _const_spec((D // 2, OUT)), _const_spec((1, OUT)),
    ],
    out_specs=_const_spec((G, OUT)),
    out_shape=jax.ShapeDtypeStruct((G, OUT), jnp.float32),
    scratch_shapes=[
        pltpu.VMEM((G, D), jnp.float32),
        pltpu.VMEM((G, 1), jnp.float32),
    ],
)


@jax.jit
def kernel(x, edge_index, batch, W_in, b_in, Wc0, bc0, Wc1, bc1, Wc2, bc2,
           Wh1, bh1, Wh2, bh2):
    pad_idx = (N + (jnp.arange(EPAD - E, dtype=jnp.int32) % (NP - N)))
    src = jnp.concatenate([edge_index[0], pad_idx]).reshape(NW, CH, CL)
    dst = jnp.concatenate([edge_index[1], pad_idx]).reshape(NW, CH, CL)
    x_pad = jnp.pad(x, ((0, NP - N), (0, 0)))
    bat = jnp.pad(batch, (0, NP - N), constant_values=G).reshape(1, NP)
    ones16 = jnp.ones((NP, 16), jnp.float32)

    d = _sc_deg()(ones16, dst)
    u0, dis = _tc0(x_pad, d[0], d[1], W_in, b_in.reshape(1, D), Wc0)

    p = _sc_agg(D)(u0, src, dst)
    u1 = _tcmid(p[0], p[1], u0, dis, bc0.reshape(1, D), Wc1)
    p = _sc_agg(D)(u1, src, dst)
    u2 = _tcmid(p[0], p[1], u1, dis, bc1.reshape(1, D), Wc2)
    p = _sc_agg(D)(u2, src, dst)
    logits = _tcfinal(p[0], p[1], u2, dis, bc2.reshape(1, D), bat,
                      Wh1, bh1.reshape(1, D // 2), Wh2, bh2.reshape(1, OUT))
    return logits


# speedup vs baseline: 19.7662x; 1.0834x over previous
import functools

import jax
import jax.numpy as jnp
from jax import lax
from jax.experimental import pallas as pl
from jax.experimental.pallas import tpu as pltpu
from jax.experimental.pallas import tpu_sc as plsc

N = 10000
D = 128
G = 64
OUT = 10
E = 320000

NC = 2
NS = 16
NW = NC * NS
CL = 128
CH = 80
EPAD = NW * CH * CL
NP = 10240
RPW = NP // NS

_R = 2048
_GRID = NP // _R


def _sc_agg_body(width, u_hbm, src_hbm, dst_hbm, p_hbm,
                 s0, d0, s1, d1, r0, r1, acc, gs0, gs1, ss0, ss1, dsem):
    c = lax.axis_index("c")
    s = lax.axis_index("s")
    w = c * NS + s

    seeds = [pltpu.async_copy(u_hbm.at[pl.ds(s * RPW + i * CL, CL)],
                              acc.at[pl.ds(s * RPW + i * CL, CL)], dsem)
             for i in range(RPW // CL)]
    for cp in seeds:
        cp.wait()
    plsc.subcore_barrier()

    pltpu.sync_copy(src_hbm.at[w, 0], s0)
    pltpu.sync_copy(dst_hbm.at[w, 0], d0)

    def _pair(k, _):
        j = 2 * k
        c0 = pltpu.async_copy(u_hbm.at[s0], r0, gs0)
        pltpu.sync_copy(src_hbm.at[w, j + 1], s1)
        pltpu.sync_copy(dst_hbm.at[w, j + 1], d1)
        c1 = pltpu.async_copy(u_hbm.at[s1], r1, gs1)
        c0.wait()
        sc0 = pltpu.async_copy(r0, acc.at[d0], ss0, add=True)
        c1.wait()
        sc1 = pltpu.async_copy(r1, acc.at[d1], ss1, add=True)
        sc0.wait()
        jn = jnp.minimum(j + 2, CH - 1)
        pltpu.sync_copy(src_hbm.at[w, jn], s0)
        pltpu.sync_copy(dst_hbm.at[w, jn], d0)
        sc1.wait()
        return ()
    lax.fori_loop(0, CH // 2, _pair, ())

    plsc.subcore_barrier()

    drains = [pltpu.async_copy(acc.at[pl.ds(s * RPW + i * CL, CL)],
                               p_hbm.at[c, pl.ds(s * RPW + i * CL, CL)], dsem)
              for i in range(RPW // CL)]
    for cp in drains:
        cp.wait()


def _sc_deg_body(ones_hbm, dst_hbm, p_hbm, d0, d1, ones_t, acc, is0, is1, dsem):
    c = lax.axis_index("c")
    s = lax.axis_index("s")
    w = c * NS + s

    seeds = [pltpu.async_copy(ones_hbm.at[pl.ds(s * RPW + i * CL, CL)],
                              acc.at[pl.ds(s * RPW + i * CL, CL)], dsem)
             for i in range(RPW // CL)]
    pltpu.sync_copy(ones_hbm.at[pl.ds(0, CL)], ones_t)
    for cp in seeds:
        cp.wait()
    plsc.subcore_barrier()

    pltpu.sync_copy(dst_hbm.at[w, 0], d0)

    def _pair(k, _):
        j = 2 * k
        i1 = pltpu.async_copy(dst_hbm.at[w, j + 1], d1, is1)
        pltpu.sync_copy(ones_t, acc.at[d0], add=True)
        i1.wait()
        jn = jnp.minimum(j + 2, CH - 1)
        i0 = pltpu.async_copy(dst_hbm.at[w, jn], d0, is0)
        pltpu.sync_copy(ones_t, acc.at[d1], add=True)
        i0.wait()
        return ()
    lax.fori_loop(0, CH // 2, _pair, ())

    plsc.subcore_barrier()

    drains = [pltpu.async_copy(acc.at[pl.ds(s * RPW + i * CL, CL)],
                               p_hbm.at[c, pl.ds(s * RPW + i * CL, CL)], dsem)
              for i in range(RPW // CL)]
    for cp in drains:
        cp.wait()


@functools.cache
def _sc_mesh():
    return plsc.VectorSubcoreMesh(core_axis_name="c", subcore_axis_name="s",
                                  num_cores=NC, num_subcores=NS)


@functools.cache
def _sc_deg():
    return pl.kernel(
        _sc_deg_body,
        out_type=jax.ShapeDtypeStruct((NC, NP, 16), jnp.float32),
        mesh=_sc_mesh(),
        scratch_types=[
            pltpu.VMEM((CL,), jnp.int32),
            pltpu.VMEM((CL,), jnp.int32),
            pltpu.VMEM((CL, 16), jnp.float32),
            pltpu.VMEM_SHARED((NP, 16), jnp.float32),
            pltpu.SemaphoreType.DMA,
            pltpu.SemaphoreType.DMA,
            pltpu.SemaphoreType.DMA,
        ],
    )


@functools.cache
def _sc_agg(width):
    return pl.kernel(
        functools.partial(_sc_agg_body, width),
        out_type=jax.ShapeDtypeStruct((NC, NP, width), jnp.float32),
        mesh=_sc_mesh(),
        scratch_types=[
            pltpu.VMEM((CL,), jnp.int32),
            pltpu.VMEM((CL,), jnp.int32),
            pltpu.VMEM((CL,), jnp.int32),
            pltpu.VMEM((CL,), jnp.int32),
            pltpu.VMEM((CL, width), jnp.float32),
            pltpu.VMEM((CL, width), jnp.float32),
            pltpu.VMEM_SHARED((NP, width), jnp.float32),
            pltpu.SemaphoreType.DMA,
            pltpu.SemaphoreType.DMA,
            pltpu.SemaphoreType.DMA,
            pltpu.SemaphoreType.DMA,
            pltpu.SemaphoreType.DMA,
        ],
    )


def _tc0_body(x_ref, d0_ref, d1_ref, win_ref, bin_ref, wc0_ref, u0_ref, dis_ref):
    deg = d0_ref[:, :1] + d1_ref[:, :1] - 1.0
    dis = lax.rsqrt(deg)
    dis_ref[...] = dis
    h0 = jnp.dot(x_ref[...], win_ref[...],
                 preferred_element_type=jnp.float32) + bin_ref[...]
    u0_ref[...] = dis * jnp.dot(h0, wc0_ref[...],
                                preferred_element_type=jnp.float32)


def _tcmid_body(p0_ref, p1_ref, up_ref, dis_ref, b_ref, w_ref, u_ref):
    dis = dis_ref[...]
    h = jnp.maximum(
        dis * (p0_ref[...] + p1_ref[...] - up_ref[...]) + b_ref[...], 0.0)
    u_ref[...] = dis * jnp.dot(h, w_ref[...],
                               preferred_element_type=jnp.float32)


def _tcfinal_body(p0_ref, p1_ref, up_ref, dis_ref, b_ref, bat_ref,
                  wh1_ref, bh1_ref, wh2_ref, bh2_ref, out_ref, sums, cnts):
    i = pl.program_id(0)

    @pl.when(i == 0)
    def _():
        sums[...] = jnp.zeros_like(sums)
        cnts[...] = jnp.zeros_like(cnts)

    dis = dis_ref[...]
    h = jnp.maximum(
        dis * (p0_ref[...] + p1_ref[...] - up_ref[...]) + b_ref[...], 0.0)
    gids = lax.broadcasted_iota(jnp.int32, (G, 1), 0)
    mask = (bat_ref[...] == gids).astype(jnp.float32)
    sums[...] += jnp.dot(mask, h, preferred_element_type=jnp.float32)
    cnts[...] += jnp.sum(mask, axis=1, keepdims=True)

    @pl.when(i == _GRID - 1)
    def _():
        g = sums[...] / jnp.maximum(cnts[...], 1.0)
        z = jnp.maximum(jnp.dot(g, wh1_ref[...],
                                preferred_element_type=jnp.float32)
                        + bh1_ref[...], 0.0)
        out_ref[...] = jnp.dot(z, wh2_ref[...],
                               preferred_element_type=jnp.float32) + bh2_ref[...]


def _row_spec(cols):
    return pl.BlockSpec((_R, cols), lambda i: (i, 0))


def _const_spec(shape):
    return pl.BlockSpec(shape, lambda i: tuple(0 for _ in shape))


_tc0 = pl.pallas_call(
    _tc0_body,
    grid=(_GRID,),
    in_specs=[
        _row_spec(D), _row_spec(16), _row_spec(16),
        _const_spec((D, D)), _const_spec((1, D)), _const_spec((D, D)),
    ],
    out_specs=[_row_spec(D), _row_spec(1)],
    out_shape=[
        jax.ShapeDtypeStruct((NP, D), jnp.float32),
        jax.ShapeDtypeStruct((NP, 1), jnp.float32),
    ],
)

_tcmid = pl.pallas_call(
    _tcmid_body,
    grid=(_GRID,),
    in_specs=[
        _row_spec(D), _row_spec(D), _row_spec(D), _row_spec(1),
        _const_spec((1, D)), _const_spec((D, D)),
    ],
    out_specs=_row_spec(D),
    out_shape=jax.ShapeDtypeStruct((NP, D), jnp.float32),
)

_tcfinal = pl.pallas_call(
    _tcfinal_body,
    grid=(_GRID,),
    in_specs=[
        _row_spec(D), _row_spec(D), _row_spec(D), _row_spec(1),
        _const_spec((1, D)), pl.BlockSpec((1, _R), lambda i: (0, i)),
        _const_spec((D, D // 2)), _const_spec((1, D // 2)),
        _const_spec((D // 2, OUT)), _const_spec((1, OUT)),
    ],
    out_specs=_const_spec((G, OUT)),
    out_shape=jax.ShapeDtypeStruct((G, OUT), jnp.float32),
    scratch_shapes=[
        pltpu.VMEM((G, D), jnp.float32),
        pltpu.VMEM((G, 1), jnp.float32),
    ],
)


@jax.jit
def kernel(x, edge_index, batch, W_in, b_in, Wc0, bc0, Wc1, bc1, Wc2, bc2,
           Wh1, bh1, Wh2, bh2):
    pad_idx = (N + (jnp.arange(EPAD - E, dtype=jnp.int32) % (NP - N)))
    src = jnp.concatenate([edge_index[0], pad_idx]).reshape(NW, CH, CL)
    dst = jnp.concatenate([edge_index[1], pad_idx]).reshape(NW, CH, CL)
    x_pad = jnp.pad(x, ((0, NP - N), (0, 0)))
    bat = jnp.pad(batch, (0, NP - N), constant_values=G).reshape(1, NP)
    ones16 = jnp.ones((NP, 16), jnp.float32)

    d = _sc_deg()(ones16, dst)
    u0, dis = _tc0(x_pad, d[0], d[1], W_in, b_in.reshape(1, D), Wc0)

    p = _sc_agg(D)(u0, src, dst)
    u1 = _tcmid(p[0], p[1], u0, dis, bc0.reshape(1, D), Wc1)
    p = _sc_agg(D)(u1, src, dst)
    u2 = _tcmid(p[0], p[1], u1, dis, bc1.reshape(1, D), Wc2)
    p = _sc_agg(D)(u2, src, dst)
    logits = _tcfinal(p[0], p[1], u2, dis, bc2.reshape(1, D), bat,
                      Wh1, bh1.reshape(1, D // 2), Wh2, bh2.reshape(1, OUT))
    return logits


# speedup vs baseline: 20.8867x; 1.0567x over previous
import functools

import jax
import jax.numpy as jnp
from jax import lax
from jax.experimental import pallas as pl
from jax.experimental.pallas import tpu as pltpu
from jax.experimental.pallas import tpu_sc as plsc

N = 10000
D = 128
G = 64
OUT = 10
E = 320000

NC = 2
NS = 16
NW = NC * NS
CL = 128
CH = 80
EPAD = NW * CH * CL
NP = 10240
RPW = NP // NS

_R = 2048
_GRID = NP // _R


def _sc_agg_body(width, u_hbm, src_hbm, dst_hbm, p_hbm,
                 s0, d0, s1, d1, r0, r1, acc, gs0, gs1, ss0, ss1, dsem):
    c = lax.axis_index("c")
    s = lax.axis_index("s")
    w = c * NS + s

    seeds = [pltpu.async_copy(u_hbm.at[pl.ds(s * RPW + i * CL, CL)],
                              acc.at[pl.ds(s * RPW + i * CL, CL)], dsem)
             for i in range(RPW // CL)]
    for cp in seeds:
        cp.wait()
    plsc.subcore_barrier()

    pltpu.sync_copy(src_hbm.at[w, 0], s0)
    pltpu.sync_copy(dst_hbm.at[w, 0], d0)

    def _pair(k, _):
        j = 2 * k
        c0 = pltpu.async_copy(u_hbm.at[s0], r0, gs0)
        pltpu.sync_copy(src_hbm.at[w, j + 1], s1)
        pltpu.sync_copy(dst_hbm.at[w, j + 1], d1)
        c1 = pltpu.async_copy(u_hbm.at[s1], r1, gs1)
        c0.wait()
        sc0 = pltpu.async_copy(r0, acc.at[d0], ss0, add=True)
        c1.wait()
        sc1 = pltpu.async_copy(r1, acc.at[d1], ss1, add=True)
        sc0.wait()
        jn = jnp.minimum(j + 2, CH - 1)
        i0 = pltpu.async_copy(src_hbm.at[w, jn], s0, gs0)
        i1 = pltpu.async_copy(dst_hbm.at[w, jn], d0, gs1)
        sc1.wait()
        i0.wait()
        i1.wait()
        return ()
    lax.fori_loop(0, CH // 2, _pair, ())

    plsc.subcore_barrier()

    drains = [pltpu.async_copy(acc.at[pl.ds(s * RPW + i * CL, CL)],
                               p_hbm.at[c, pl.ds(s * RPW + i * CL, CL)], dsem)
              for i in range(RPW // CL)]
    for cp in drains:
        cp.wait()


def _sc_deg_body(ones_hbm, dst_hbm, p_hbm, d0, d1, ones_t, acc, is0, is1, dsem):
    c = lax.axis_index("c")
    s = lax.axis_index("s")
    w = c * NS + s

    seeds = [pltpu.async_copy(ones_hbm.at[pl.ds(s * RPW + i * CL, CL)],
                              acc.at[pl.ds(s * RPW + i * CL, CL)], dsem)
             for i in range(RPW // CL)]
    pltpu.sync_copy(ones_hbm.at[pl.ds(0, CL)], ones_t)
    for cp in seeds:
        cp.wait()
    plsc.subcore_barrier()

    pltpu.sync_copy(dst_hbm.at[w, 0], d0)

    def _pair(k, _):
        j = 2 * k
        i1 = pltpu.async_copy(dst_hbm.at[w, j + 1], d1, is1)
        pltpu.sync_copy(ones_t, acc.at[d0], add=True)
        i1.wait()
        jn = jnp.minimum(j + 2, CH - 1)
        i0 = pltpu.async_copy(dst_hbm.at[w, jn], d0, is0)
        pltpu.sync_copy(ones_t, acc.at[d1], add=True)
        i0.wait()
        return ()
    lax.fori_loop(0, CH // 2, _pair, ())

    plsc.subcore_barrier()

    drains = [pltpu.async_copy(acc.at[pl.ds(s * RPW + i * CL, CL)],
                               p_hbm.at[c, pl.ds(s * RPW + i * CL, CL)], dsem)
              for i in range(RPW // CL)]
    for cp in drains:
        cp.wait()


@functools.cache
def _sc_mesh():
    return plsc.VectorSubcoreMesh(core_axis_name="c", subcore_axis_name="s",
                                  num_cores=NC, num_subcores=NS)


@functools.cache
def _sc_deg():
    return pl.kernel(
        _sc_deg_body,
        out_type=jax.ShapeDtypeStruct((NC, NP, 16), jnp.float32),
        mesh=_sc_mesh(),
        scratch_types=[
            pltpu.VMEM((CL,), jnp.int32),
            pltpu.VMEM((CL,), jnp.int32),
            pltpu.VMEM((CL, 16), jnp.float32),
            pltpu.VMEM_SHARED((NP, 16), jnp.float32),
            pltpu.SemaphoreType.DMA,
            pltpu.SemaphoreType.DMA,
            pltpu.SemaphoreType.DMA,
        ],
    )


@functools.cache
def _sc_agg(width):
    return pl.kernel(
        functools.partial(_sc_agg_body, width),
        out_type=jax.ShapeDtypeStruct((NC, NP, width), jnp.float32),
        mesh=_sc_mesh(),
        scratch_types=[
            pltpu.VMEM((CL,), jnp.int32),
            pltpu.VMEM((CL,), jnp.int32),
            pltpu.VMEM((CL,), jnp.int32),
            pltpu.VMEM((CL,), jnp.int32),
            pltpu.VMEM((CL, width), jnp.float32),
            pltpu.VMEM((CL, width), jnp.float32),
            pltpu.VMEM_SHARED((NP, width), jnp.float32),
            pltpu.SemaphoreType.DMA,
            pltpu.SemaphoreType.DMA,
            pltpu.SemaphoreType.DMA,
            pltpu.SemaphoreType.DMA,
            pltpu.SemaphoreType.DMA,
        ],
    )


def _tc0a_body(x_ref, win_ref, bin_ref, wc0_ref, h0c_ref):
    h0 = jnp.dot(x_ref[...], win_ref[...],
                 preferred_element_type=jnp.float32) + bin_ref[...]
    h0c_ref[...] = jnp.dot(h0, wc0_ref[...],
                           preferred_element_type=jnp.float32)


def _tc0b_body(h0c_ref, d0_ref, d1_ref, u0_ref, dis_ref):
    deg = d0_ref[:, :1] + d1_ref[:, :1] - 1.0
    dis = lax.rsqrt(deg)
    dis_ref[...] = dis
    u0_ref[...] = dis * h0c_ref[...]


def _tcmid_body(p0_ref, p1_ref, up_ref, dis_ref, b_ref, w_ref, u_ref):
    dis = dis_ref[...]
    h = jnp.maximum(
        dis * (p0_ref[...] + p1_ref[...] - up_ref[...]) + b_ref[...], 0.0)
    u_ref[...] = dis * jnp.dot(h, w_ref[...],
                               preferred_element_type=jnp.float32)


def _tcfinal_body(p0_ref, p1_ref, up_ref, dis_ref, b_ref, bat_ref,
                  wh1_ref, bh1_ref, wh2_ref, bh2_ref, out_ref, sums, cnts):
    i = pl.program_id(0)

    @pl.when(i == 0)
    def _():
        sums[...] = jnp.zeros_like(sums)
        cnts[...] = jnp.zeros_like(cnts)

    dis = dis_ref[...]
    h = jnp.maximum(
        dis * (p0_ref[...] + p1_ref[...] - up_ref[...]) + b_ref[...], 0.0)
    gids = lax.broadcasted_iota(jnp.int32, (G, 1), 0)
    mask = (bat_ref[...] == gids).astype(jnp.float32)
    sums[...] += jnp.dot(mask, h, preferred_element_type=jnp.float32)
    cnts[...] += jnp.sum(mask, axis=1, keepdims=True)

    @pl.when(i == _GRID - 1)
    def _():
        g = sums[...] / jnp.maximum(cnts[...], 1.0)
        z = jnp.maximum(jnp.dot(g, wh1_ref[...],
                                preferred_element_type=jnp.float32)
                        + bh1_ref[...], 0.0)
        out_ref[...] = jnp.dot(z, wh2_ref[...],
                               preferred_element_type=jnp.float32) + bh2_ref[...]


def _row_spec(cols):
    return pl.BlockSpec((_R, cols), lambda i: (i, 0))


def _const_spec(shape):
    return pl.BlockSpec(shape, lambda i: tuple(0 for _ in shape))


_tc0a = pl.pallas_call(
    _tc0a_body,
    grid=(_GRID,),
    in_specs=[
        _row_spec(D),
        _const_spec((D, D)), _const_spec((1, D)), _const_spec((D, D)),
    ],
    out_specs=_row_spec(D),
    out_shape=jax.ShapeDtypeStruct((NP, D), jnp.float32),
)

_tc0b = pl.pallas_call(
    _tc0b_body,
    grid=(_GRID,),
    in_specs=[_row_spec(D), _row_spec(16), _row_spec(16)],
    out_specs=[_row_spec(D), _row_spec(1)],
    out_shape=[
        jax.ShapeDtypeStruct((NP, D), jnp.float32),
        jax.ShapeDtypeStruct((NP, 1), jnp.float32),
    ],
)

_tcmid = pl.pallas_call(
    _tcmid_body,
    grid=(_GRID,),
    in_specs=[
        _row_spec(D), _row_spec(D), _row_spec(D), _row_spec(1),
        _const_spec((1, D)), _const_spec((D, D)),
    ],
    out_specs=_row_spec(D),
    out_shape=jax.ShapeDtypeStruct((NP, D), jnp.float32),
)

_tcfinal = pl.pallas_call(
    _tcfinal_body,
    grid=(_GRID,),
    in_specs=[
        _row_spec(D), _row_spec(D), _row_spec(D), _row_spec(1),
        _const_spec((1, D)), pl.BlockSpec((1, _R), lambda i: (0, i)),
        _const_spec((D, D // 2)), _const_spec((1, D // 2)),
        _const_spec((D // 2, OUT)), _const_spec((1, OUT)),
    ],
    out_specs=_const_spec((G, OUT)),
    out_shape=jax.ShapeDtypeStruct((G, OUT), jnp.float32),
    scratch_shapes=[
        pltpu.VMEM((G, D), jnp.float32),
        pltpu.VMEM((G, 1), jnp.float32),
    ],
)


@jax.jit
def kernel(x, edge_index, batch, W_in, b_in, Wc0, bc0, Wc1, bc1, Wc2, bc2,
           Wh1, bh1, Wh2, bh2):
    pad_idx = (N + (jnp.arange(EPAD - E, dtype=jnp.int32) % (NP - N)))
    src = jnp.concatenate([edge_index[0], pad_idx]).reshape(NW, CH, CL)
    dst = jnp.concatenate([edge_index[1], pad_idx]).reshape(NW, CH, CL)
    x_pad = jnp.pad(x, ((0, NP - N), (0, 0)))
    bat = jnp.pad(batch, (0, NP - N), constant_values=G).reshape(1, NP)
    ones16 = jnp.ones((NP, 16), jnp.float32)

    d = _sc_deg()(ones16, dst)
    h0c = _tc0a(x_pad, W_in, b_in.reshape(1, D), Wc0)
    u0, dis = _tc0b(h0c, d[0], d[1])

    p = _sc_agg(D)(u0, src, dst)
    u1 = _tcmid(p[0], p[1], u0, dis, bc0.reshape(1, D), Wc1)
    p = _sc_agg(D)(u1, src, dst)
    u2 = _tcmid(p[0], p[1], u1, dis, bc1.reshape(1, D), Wc2)
    p = _sc_agg(D)(u2, src, dst)
    logits = _tcfinal(p[0], p[1], u2, dis, bc2.reshape(1, D), bat,
                      Wh1, bh1.reshape(1, D // 2), Wh2, bh2.reshape(1, OUT))
    return logits
